# baseline (speedup 1.0000x reference)
import jax
import jax.numpy as jnp
from jax.experimental import pallas as pl
from jax.experimental.pallas import tpu as pltpu

_BLOCK_ROWS = 1000


def _body(h_blk, o_blk, w, bias):
    x = h_blk[...].astype(jnp.bfloat16)
    acc = jax.lax.dot_general(
        x, w, (((1,), (1,)), ((), ())),
        preferred_element_type=jnp.float32,
    )
    o_blk[...] = jnp.maximum(acc + bias, 0.0)


def _outer(h_hbm, w_ref, b_ref, o_hbm):
    n = h_hbm.shape[0]
    d_in = h_hbm.shape[1]
    d_out = o_hbm.shape[1]
    w = w_ref[...].astype(jnp.bfloat16)
    bias = b_ref[...]
    pipe = pltpu.emit_pipeline(
        lambda h_blk, o_blk: _body(h_blk, o_blk, w, bias),
        grid=(n // _BLOCK_ROWS,),
        in_specs=[pl.BlockSpec((_BLOCK_ROWS, d_in), lambda i: (i, 0))],
        out_specs=[pl.BlockSpec((_BLOCK_ROWS, d_out), lambda i: (i, 0))],
    )
    pipe(h_hbm, o_hbm)


def kernel(h, edge_index, W, b):
    del edge_index
    n, d_in = h.shape
    d_out = W.shape[0]
    b2 = b.reshape(1, d_out)
    return pl.pallas_call(
        _outer,
        in_specs=[
            pl.BlockSpec(memory_space=pl.ANY),
            pl.BlockSpec(memory_space=pltpu.MemorySpace.VMEM),
            pl.BlockSpec(memory_space=pltpu.MemorySpace.VMEM),
        ],
        out_specs=pl.BlockSpec(memory_space=pl.ANY),
        out_shape=jax.ShapeDtypeStruct((n, d_out), jnp.float32),
    )(h, W, b2)


# speedup vs baseline: 2.0224x; 2.0224x over previous
import jax
import jax.numpy as jnp
from jax.experimental import pallas as pl
from jax.experimental.pallas import tpu as pltpu

_BLOCK_ROWS = 5000


def _fused_linear_relu(h_ref, w_ref, b_ref, o_ref):
    x = h_ref[...].astype(jnp.bfloat16)
    acc = jax.lax.dot_general(
        x, w_ref[...].astype(jnp.bfloat16), (((1,), (1,)), ((), ())),
        preferred_element_type=jnp.float32,
    )
    o_ref[...] = jnp.maximum(acc + b_ref[...], 0.0)


def kernel(h, edge_index, W, b):
    del edge_index
    n, d_in = h.shape
    d_out = W.shape[0]
    b2 = b.reshape(1, d_out)
    return pl.pallas_call(
        _fused_linear_relu,
        grid=(pl.cdiv(n, _BLOCK_ROWS),),
        in_specs=[
            pl.BlockSpec((_BLOCK_ROWS, d_in), lambda i: (i, 0)),
            pl.BlockSpec((d_out, d_in), lambda i: (0, 0)),
            pl.BlockSpec((1, d_out), lambda i: (0, 0)),
        ],
        out_specs=pl.BlockSpec((_BLOCK_ROWS, d_out), lambda i: (i, 0)),
        out_shape=jax.ShapeDtypeStruct((n, d_out), jnp.float32),
        compiler_params=pltpu.CompilerParams(
            dimension_semantics=("arbitrary",),
            disable_bounds_checks=True,
            disable_semaphore_checks=True,
            skip_device_barrier=True,
        ),
    )(h, W, b2)
